# baseline (speedup 1.0000x reference)
import jax
import jax.numpy as jnp
from jax import lax
from jax.experimental import pallas as pl
from jax.experimental.pallas import tpu as pltpu
from jax.experimental.pallas import tpu_sc as plsc

_BATCH = 4096
_N_FIELDS = 26
_NF_PAD = 32
_PER_FIELD_VOCAB = 50
_EMBED_DIM = 64
_PAD_DIM = 128

_NC = 2
_NS = 16
_NW = _NC * _NS

_COLS_W = _BATCH // _NW
_LANES = 16


def _body(cat_hbm, table_hbm, out_hbm, cat_t, idx_t, bg, bt, gsems, wsems):
    wid = lax.axis_index("s") * _NC + lax.axis_index("c")
    col0 = wid * _COLS_W

    pltpu.sync_copy(cat_hbm.at[:, pl.ds(col0, _COLS_W)], cat_t)

    for f in range(_N_FIELDS):
        for kb in range(_COLS_W // _LANES):
            s = kb * _LANES
            idx_t[f, pl.ds(s, _LANES)] = cat_t[f, pl.ds(s, _LANES)] + (
                f * _PER_FIELD_VOCAB
            )

    def gather(f, q):
        pltpu.async_copy(table_hbm.at[idx_t.at[f]], bg[q], gsems[q])

    def gather_wait(f, q):
        pltpu.make_async_copy(table_hbm.at[idx_t.at[f]], bg[q], gsems[q]).wait()

    def write(f, q):
        pltpu.async_copy(
            bt[q].at[:, pl.ds(0, _COLS_W)],
            out_hbm.at[f, :, pl.ds(col0, _COLS_W)],
            wsems[q],
        )

    def write_wait(f, q):
        pltpu.make_async_copy(
            bt[q].at[:, pl.ds(0, _COLS_W)],
            out_hbm.at[f, :, pl.ds(col0, _COLS_W)],
            wsems[q],
        ).wait()

    gather(0, 0)
    gather(1, 1)

    evc = [
        (lax.broadcasted_iota(jnp.int32, (_LANES,), 0) + ke * _LANES)
        for ke in range(_EMBED_DIM // _LANES)
    ]

    @pl.loop(0, _N_FIELDS // 2)
    def _f_loop(j):
        for q in range(2):
            f = 2 * j + q
            gather_wait(f, q)
            @pl.when(f >= 2)
            def _():
                write_wait(f - 2, q)

            @plsc.parallel_loop(0, _COLS_W, unroll=4)
            def _b_loop(b):
                bv = jnp.full((_LANES,), b, dtype=jnp.int32)
                for ke in range(_EMBED_DIM // _LANES):
                    v = bg[q][b, pl.ds(ke * _LANES, _LANES)]
                    plsc.store_scatter(bt[q], [evc[ke], bv], v)

            write(f, q)

            @pl.when(f + 2 < _N_FIELDS)
            def _():
                gather(f + 2, q)

    write_wait(_N_FIELDS - 2, 0)
    write_wait(_N_FIELDS - 1, 1)


@jax.jit
def _lookup(cat_t_pad, table_pad):
    mesh = plsc.VectorSubcoreMesh(
        core_axis_name="c", subcore_axis_name="s", num_cores=_NC, num_subcores=_NS
    )
    k = pl.kernel(
        _body,
        out_type=jax.ShapeDtypeStruct((_N_FIELDS, _EMBED_DIM, _BATCH), jnp.float32),
        mesh=mesh,
        scratch_types=[
            pltpu.VMEM((_NF_PAD, _COLS_W), jnp.int32),
            pltpu.VMEM((_N_FIELDS, _COLS_W), jnp.int32),
            [pltpu.VMEM((_COLS_W, _PAD_DIM), jnp.float32) for _ in range(2)],
            [pltpu.VMEM((_EMBED_DIM, _COLS_W + 1), jnp.float32) for _ in range(2)],
            [pltpu.SemaphoreType.DMA for _ in range(2)],
            [pltpu.SemaphoreType.DMA for _ in range(2)],
        ],
        compiler_params=pltpu.CompilerParams(needs_layout_passes=False),
    )
    return k(cat_t_pad, table_pad)


def kernel(cat, table):
    cat_t_pad = jnp.pad(cat.T, ((0, _NF_PAD - _N_FIELDS), (0, 0)))
    table_pad = jnp.pad(table, ((0, 0), (0, _PAD_DIM - _EMBED_DIM)))
    out_t = _lookup(cat_t_pad, table_pad)
    return jnp.transpose(out_t, (2, 0, 1))


# speedup vs baseline: 1.2880x; 1.2880x over previous
import jax
import jax.numpy as jnp
from jax import lax
from jax.experimental import pallas as pl
from jax.experimental.pallas import tpu as pltpu
from jax.experimental.pallas import tpu_sc as plsc

_BATCH = 4096
_N_FIELDS = 26
_NF_PAD = 32
_PER_FIELD_VOCAB = 50
_EMBED_DIM = 64
_PAD_DIM = 128

_NC = 2
_NS = 16
_NW = _NC * _NS

_COLS_W = _BATCH // _NW
_LANES = 16


def _body(cat_hbm, table_hbm, out_hbm, cat_t, idx_t, bg, bt, gsems, wsems):
    wid = lax.axis_index("s") * _NC + lax.axis_index("c")
    col0 = wid * _COLS_W

    pltpu.sync_copy(cat_hbm.at[:, pl.ds(col0, _COLS_W)], cat_t)

    for f in range(_N_FIELDS):
        for kb in range(_COLS_W // _LANES):
            s = kb * _LANES
            idx_t[f, pl.ds(s, _LANES)] = cat_t[f, pl.ds(s, _LANES)] + (
                f * _PER_FIELD_VOCAB
            )

    def gather(f, q):
        pltpu.async_copy(table_hbm.at[idx_t.at[f]], bg[q], gsems[q])

    def gather_wait(f, q):
        pltpu.make_async_copy(table_hbm.at[idx_t.at[f]], bg[q], gsems[q]).wait()

    def write(f, q):
        pltpu.async_copy(bt[q], out_hbm.at[f, :, pl.ds(col0, _COLS_W)], wsems[q])

    def write_wait(f, q):
        pltpu.make_async_copy(
            bt[q], out_hbm.at[f, :, pl.ds(col0, _COLS_W)], wsems[q]
        ).wait()

    gather(0, 0)
    gather(1, 1)

    base = [
        (lax.broadcasted_iota(jnp.int32, (_LANES,), 0) + kb * _LANES)
        for kb in range(_COLS_W // _LANES)
    ]

    @pl.loop(0, _N_FIELDS // 2)
    def _f_loop(j):
        for q in range(2):
            f = 2 * j + q
            gather_wait(f, q)
            @pl.when(f >= 2)
            def _():
                write_wait(f - 2, q)

            @plsc.parallel_loop(0, _LANES, unroll=8)
            def _d_loop(d):
                rot = lax.rem(
                    lax.broadcasted_iota(jnp.int32, (_LANES,), 0) + d, _LANES
                )
                for ke in range(_EMBED_DIM // _LANES):
                    ev = rot + ke * _LANES
                    for kb in range(_COLS_W // _LANES):
                        v = plsc.load_gather(bg[q], [base[kb], ev])
                        plsc.store_scatter(bt[q], [ev, base[kb]], v)

            write(f, q)

            @pl.when(f + 2 < _N_FIELDS)
            def _():
                gather(f + 2, q)

    write_wait(_N_FIELDS - 2, 0)
    write_wait(_N_FIELDS - 1, 1)


@jax.jit
def _lookup(cat_t_pad, table_pad):
    mesh = plsc.VectorSubcoreMesh(
        core_axis_name="c", subcore_axis_name="s", num_cores=_NC, num_subcores=_NS
    )
    k = pl.kernel(
        _body,
        out_type=jax.ShapeDtypeStruct((_N_FIELDS, _EMBED_DIM, _BATCH), jnp.float32),
        mesh=mesh,
        scratch_types=[
            pltpu.VMEM((_NF_PAD, _COLS_W), jnp.int32),
            pltpu.VMEM((_N_FIELDS, _COLS_W), jnp.int32),
            [pltpu.VMEM((_COLS_W, _PAD_DIM), jnp.float32) for _ in range(2)],
            [pltpu.VMEM((_EMBED_DIM, _COLS_W), jnp.float32) for _ in range(2)],
            [pltpu.SemaphoreType.DMA for _ in range(2)],
            [pltpu.SemaphoreType.DMA for _ in range(2)],
        ],
        compiler_params=pltpu.CompilerParams(needs_layout_passes=False),
    )
    return k(cat_t_pad, table_pad)


def kernel(cat, table):
    cat_t_pad = jnp.pad(cat.T, ((0, _NF_PAD - _N_FIELDS), (0, 0)))
    table_pad = jnp.pad(table, ((0, 0), (0, _PAD_DIM - _EMBED_DIM)))
    out_t = _lookup(cat_t_pad, table_pad)
    return jnp.transpose(out_t, (2, 0, 1))


# speedup vs baseline: 1.3373x; 1.0383x over previous
import jax
import jax.numpy as jnp
from jax import lax
from jax.experimental import pallas as pl
from jax.experimental.pallas import tpu as pltpu
from jax.experimental.pallas import tpu_sc as plsc

_BATCH = 4096
_N_FIELDS = 26
_NF_PAD = 32
_PER_FIELD_VOCAB = 50
_EMBED_DIM = 64
_PAD_DIM = 128

_NC = 2
_NS = 16
_NW = _NC * _NS

_COLS_W = _BATCH // _NW
_LANES = 16


def _body(cat_hbm, table_hbm, out_hbm, cat_t, idx_t, bg, bt, gsems, wsems):
    wid = lax.axis_index("s") * _NC + lax.axis_index("c")
    col0 = wid * _COLS_W

    pltpu.sync_copy(cat_hbm.at[:, pl.ds(col0, _COLS_W)], cat_t)

    for f in range(_N_FIELDS):
        for kb in range(_COLS_W // _LANES):
            s = kb * _LANES
            idx_t[f, pl.ds(s, _LANES)] = cat_t[f, pl.ds(s, _LANES)] + (
                f * _PER_FIELD_VOCAB
            )

    def gather(f, q):
        pltpu.async_copy(table_hbm.at[idx_t.at[f]], bg[q], gsems[q])

    def gather_wait(f, q):
        pltpu.make_async_copy(table_hbm.at[idx_t.at[f]], bg[q], gsems[q]).wait()

    def write(f, q):
        pltpu.async_copy(bt[q], out_hbm.at[f, :, pl.ds(col0, _COLS_W)], wsems[q])

    def write_wait(f, q):
        pltpu.make_async_copy(
            bt[q], out_hbm.at[f, :, pl.ds(col0, _COLS_W)], wsems[q]
        ).wait()

    gather(0, 0)
    gather(1, 1)

    base = [
        (lax.broadcasted_iota(jnp.int32, (_LANES,), 0) + kb * _LANES)
        for kb in range(_COLS_W // _LANES)
    ]

    @pl.loop(0, _N_FIELDS // 2)
    def _f_loop(j):
        for q in range(2):
            f = 2 * j + q
            gather_wait(f, q)
            @pl.when(f >= 2)
            def _():
                write_wait(f - 2, q)

            @plsc.parallel_loop(0, _LANES, unroll=4)
            def _d_loop(d):
                rot = lax.rem(
                    lax.broadcasted_iota(jnp.int32, (_LANES,), 0) + d, _LANES
                )
                for ke in range(_EMBED_DIM // _LANES):
                    ev = rot + ke * _LANES
                    for kb in range(_COLS_W // _LANES):
                        v = plsc.load_gather(bg[q], [base[kb], ev])
                        plsc.store_scatter(bt[q], [ev, base[kb]], v)

            write(f, q)

            @pl.when(f + 2 < _N_FIELDS)
            def _():
                gather(f + 2, q)

    write_wait(_N_FIELDS - 2, 0)
    write_wait(_N_FIELDS - 1, 1)


@jax.jit
def _lookup(cat_t_pad, table_pad):
    mesh = plsc.VectorSubcoreMesh(
        core_axis_name="c", subcore_axis_name="s", num_cores=_NC, num_subcores=_NS
    )
    k = pl.kernel(
        _body,
        out_type=jax.ShapeDtypeStruct((_N_FIELDS, _EMBED_DIM, _BATCH), jnp.float32),
        mesh=mesh,
        scratch_types=[
            pltpu.VMEM((_NF_PAD, _COLS_W), jnp.int32),
            pltpu.VMEM((_N_FIELDS, _COLS_W), jnp.int32),
            [pltpu.VMEM((_COLS_W, _PAD_DIM), jnp.float32) for _ in range(2)],
            [pltpu.VMEM((_EMBED_DIM, _COLS_W), jnp.float32) for _ in range(2)],
            [pltpu.SemaphoreType.DMA for _ in range(2)],
            [pltpu.SemaphoreType.DMA for _ in range(2)],
        ],
        compiler_params=pltpu.CompilerParams(needs_layout_passes=False),
    )
    return k(cat_t_pad, table_pad)


def kernel(cat, table):
    cat_t_pad = jnp.pad(cat.T, ((0, _NF_PAD - _N_FIELDS), (0, 0)))
    table_pad = jnp.pad(table, ((0, 0), (0, _PAD_DIM - _EMBED_DIM)))
    out_t = _lookup(cat_t_pad, table_pad)
    return jnp.transpose(out_t, (2, 0, 1))
